# baseline (speedup 1.0000x reference)
import functools

import jax
import jax.numpy as jnp
from jax import lax
from jax.experimental import pallas as pl
from jax.experimental.pallas import tpu as pltpu
from jax.experimental.pallas import tpu_sc as plsc

M_SEG = 25000
NC = 2
NS = 16
NW = NC * NS
LANES = 16
IDXW = 128
RPC_A = 8
RPC_C = 4


def _pad_to(n, m):
    return ((n + m - 1) // m) * m


def _zero_shared(rows2, shared, tile_rows, tile_base, chunk):
    zb = jnp.zeros((2 * LANES,), shared.dtype)

    def zr(i, carry):
        rows2[0, i, :] = zb
        return carry

    lax.fori_loop(0, chunk, zr, 0)
    off = 0
    while off < tile_rows:
        sz = min(chunk, tile_rows - off)
        pltpu.sync_copy(rows2.at[0, pl.ds(0, sz), :],
                        shared.at[pl.ds(tile_base + off, sz), :])
        off += sz


def _pool_body(nrows_pw, mt, src2, dst2, x_hbm, sums_out, counts_out,
               sums_sh, srcbuf2, dstbuf2, rows2, counts,
               isem0, isem1, gsem0, gsem1, ssem0, ssem1):
    c = lax.axis_index("c")
    s = lax.axis_index("s")
    w = c * NS + s
    mp = counts.shape[0]
    rpc = RPC_A
    chunk = rpc * IDXW
    nchunk = nrows_pw // rpc
    isem = (isem0, isem1)
    gsem = (gsem0, gsem1)
    ssem = (ssem0, ssem1)
    z16 = jnp.zeros((LANES,), jnp.float32)
    ones = jnp.full((LANES,), 1.0, jnp.float32)

    def zero_counts(i, carry):
        counts[pl.ds(i * LANES, LANES)] = z16
        return carry

    lax.fori_loop(0, mp // LANES, zero_counts, 0)
    _zero_shared(rows2, sums_sh, mt, s * mt, chunk)

    def issue_idx(j, b):
        rb = w * nrows_pw + j * rpc
        pltpu.async_copy(src2.at[pl.ds(rb, rpc), :], srcbuf2.at[b], isem[b])
        pltpu.async_copy(dst2.at[pl.ds(rb, rpc), :], dstbuf2.at[b], isem[b])

    issue_idx(0, 0)
    plsc.subcore_barrier()

    def half(jj, b):
        j = jj * 2 + b
        nb = 1 - b
        rb = w * nrows_pw + j * rpc
        pltpu.make_async_copy(src2.at[pl.ds(rb, rpc), :], srcbuf2.at[b],
                              isem[b]).wait()
        pltpu.make_async_copy(dst2.at[pl.ds(rb, rpc), :], dstbuf2.at[b],
                              isem[b]).wait()
        for i in range(rpc):
            for l in range(IDXW // LANES):
                dv = dstbuf2[b, i, pl.ds(l * LANES, LANES)]
                plsc.addupdate_scatter(counts, [dv], ones)
        for i in range(rpc):
            pltpu.async_copy(x_hbm.at[srcbuf2.at[b, i]],
                             rows2.at[b, pl.ds(i * IDXW, IDXW), :], gsem[b])
        @pl.when(j >= 1)
        def _():
            pltpu.make_async_copy(x_hbm.at[pl.ds(0, chunk), :], rows2.at[nb],
                                  ssem[nb]).wait()
        @pl.when(j < nchunk - 1)
        def _():
            issue_idx(j + 1, nb)
        pltpu.make_async_copy(x_hbm.at[pl.ds(0, chunk), :], rows2.at[b],
                              gsem[b]).wait()
        for i in range(rpc):
            pltpu.async_copy(rows2.at[b, pl.ds(i * IDXW, IDXW), :],
                             sums_sh.at[dstbuf2.at[b, i]], ssem[b], add=True)

    def outer(jj, carry):
        half(jj, 0)
        half(jj, 1)
        return carry

    lax.fori_loop(0, nchunk // 2, outer, 0)
    pltpu.make_async_copy(x_hbm.at[pl.ds(0, chunk), :], rows2.at[1],
                          ssem[1]).wait()
    plsc.subcore_barrier()
    pltpu.sync_copy(sums_sh.at[pl.ds(s * mt, mt), :],
                    sums_out.at[c, pl.ds(s * mt, mt), :])
    pltpu.sync_copy(counts, counts_out.at[w])


def _up_body(nrows_pw, nt, mp, src2, kern2, dst2, p_hbm, up_out,
             up_sh, sbuf2, kbuf2, dbuf2, gbuf2, rows2,
             isem0, isem1, gsem0, gsem1, ssem0, ssem1):
    c = lax.axis_index("c")
    s = lax.axis_index("s")
    w = c * NS + s
    rpc = RPC_C
    chunk = rpc * IDXW
    nchunk = nrows_pw // rpc
    isem = (isem0, isem1)
    gsem = (gsem0, gsem1)
    ssem = (ssem0, ssem1)

    _zero_shared(rows2, up_sh, nt, s * nt, chunk)

    def issue_idx(j, b):
        rb = w * nrows_pw + j * rpc
        pltpu.async_copy(src2.at[pl.ds(rb, rpc), :], sbuf2.at[b], isem[b])
        pltpu.async_copy(kern2.at[pl.ds(rb, rpc), :], kbuf2.at[b], isem[b])
        pltpu.async_copy(dst2.at[pl.ds(rb, rpc), :], dbuf2.at[b], isem[b])

    issue_idx(0, 0)
    plsc.subcore_barrier()

    def half(jj, b):
        j = jj * 2 + b
        nb = 1 - b
        rb = w * nrows_pw + j * rpc
        pltpu.make_async_copy(src2.at[pl.ds(rb, rpc), :], sbuf2.at[b],
                              isem[b]).wait()
        pltpu.make_async_copy(kern2.at[pl.ds(rb, rpc), :], kbuf2.at[b],
                              isem[b]).wait()
        pltpu.make_async_copy(dst2.at[pl.ds(rb, rpc), :], dbuf2.at[b],
                              isem[b]).wait()
        for i in range(rpc):
            for l in range(IDXW // LANES):
                sv = sbuf2[b, i, pl.ds(l * LANES, LANES)]
                kv = kbuf2[b, i, pl.ds(l * LANES, LANES)]
                gbuf2[b, i, pl.ds(l * LANES, LANES)] = kv * mp + sv
        for i in range(rpc):
            pltpu.async_copy(p_hbm.at[gbuf2.at[b, i]],
                             rows2.at[b, pl.ds(i * IDXW, IDXW), :], gsem[b])

        @pl.when(j >= 1)
        def _():
            pltpu.make_async_copy(p_hbm.at[pl.ds(0, chunk), :], rows2.at[nb],
                                  ssem[nb]).wait()

        @pl.when(j < nchunk - 1)
        def _():
            issue_idx(j + 1, nb)

        pltpu.make_async_copy(p_hbm.at[pl.ds(0, chunk), :], rows2.at[b],
                              gsem[b]).wait()
        for i in range(rpc):
            pltpu.async_copy(rows2.at[b, pl.ds(i * IDXW, IDXW), :],
                             up_sh.at[dbuf2.at[b, i]], ssem[b], add=True)

    def outer(jj, carry):
        half(jj, 0)
        half(jj, 1)
        return carry

    lax.fori_loop(0, nchunk // 2, outer, 0)
    pltpu.make_async_copy(p_hbm.at[pl.ds(0, chunk), :], rows2.at[1],
                          ssem[1]).wait()
    plsc.subcore_barrier()
    base = s * nt
    off = 0
    while off < nt:
        sz = min(2048, nt - off)
        pltpu.sync_copy(up_sh.at[pl.ds(base + off, sz), :],
                        up_out.at[c, pl.ds(base + off, sz), :])
        off += sz


def _dense_body(sums_ref, counts_ref, w_ref, out_ref):
    sums = sums_ref[0].astype(jnp.float32) + sums_ref[1].astype(jnp.float32)
    cnt = jnp.sum(counts_ref[...], axis=0)
    pooled = sums / jnp.maximum(cnt, 1.0)[:, None]
    for k in range(out_ref.shape[0]):
        out_ref[k] = jnp.dot(pooled, w_ref[k],
                             preferred_element_type=jnp.float32
                             ).astype(jnp.bfloat16)


def _final_body(x_ref, up_ref, b_ref, o_ref):
    up = up_ref[0].astype(jnp.float32) + up_ref[1].astype(jnp.float32)
    o_ref[...] = x_ref[...] - up - b_ref[...]


def kernel(x, pool_src, pool_dst, up_src, up_dst, up_kernel, W, b):
    n, ch = x.shape
    e = pool_src.shape[0]
    kk = W.shape[0]
    i32 = jnp.int32

    mp = _pad_to(M_SEG + 1, 8 * NS)
    np_ = _pad_to(n + 1, 8 * NS)
    mt = mp // NS
    nt = np_ // NS
    epad = _pad_to(e, NW * 16 * IDXW)
    nrows_pw = epad // (NW * IDXW)
    pad = epad - e

    x_bf = x.astype(jnp.bfloat16)
    ps = jnp.concatenate([pool_src.astype(i32), jnp.zeros((pad,), i32)])
    pd = jnp.concatenate([pool_dst.astype(i32), jnp.full((pad,), M_SEG, i32)])
    us = jnp.concatenate([up_src.astype(i32), jnp.zeros((pad,), i32)])
    uk = jnp.concatenate([up_kernel.astype(i32), jnp.zeros((pad,), i32)])
    ud = jnp.concatenate([up_dst.astype(i32), jnp.full((pad,), n, i32)])
    src2 = ps.reshape(-1, IDXW)
    dst2 = pd.reshape(-1, IDXW)
    usrc2 = us.reshape(-1, IDXW)
    ukern2 = uk.reshape(-1, IDXW)
    udst2 = ud.reshape(-1, IDXW)

    mesh = plsc.VectorSubcoreMesh(core_axis_name="c", subcore_axis_name="s")
    sc_params = pltpu.CompilerParams(needs_layout_passes=False,
                                     use_tc_tiling_on_sc=False)
    dma_sems = [pltpu.SemaphoreType.DMA] * 6

    sums_p, counts_p = pl.kernel(
        functools.partial(_pool_body, nrows_pw, mt),
        out_type=(jax.ShapeDtypeStruct((NC, mp, ch), jnp.bfloat16),
                  jax.ShapeDtypeStruct((NW, mp), jnp.float32)),
        mesh=mesh,
        scratch_types=[
            pltpu.MemorySpace.VMEM_SHARED((mp, ch), jnp.bfloat16),
            pltpu.VMEM((2, RPC_A, IDXW), i32),
            pltpu.VMEM((2, RPC_A, IDXW), i32),
            pltpu.VMEM((2, RPC_A * IDXW, ch), jnp.bfloat16),
            pltpu.VMEM((mp,), jnp.float32),
        ] + dma_sems,
        compiler_params=sc_params,
        name="sc_pool_segment_sum",
    )(src2, dst2, x_bf)

    bm = mp // 14
    p_tab = pl.pallas_call(
        _dense_body,
        grid=(mp // bm,),
        in_specs=[
            pl.BlockSpec((NC, bm, ch), lambda j: (0, j, 0)),
            pl.BlockSpec((NW, bm), lambda j: (0, j)),
            pl.BlockSpec((kk, ch, ch), lambda j: (0, 0, 0)),
        ],
        out_specs=pl.BlockSpec((kk, bm, ch), lambda j: (0, j, 0)),
        out_shape=jax.ShapeDtypeStruct((kk, mp, ch), jnp.bfloat16),
    )(sums_p, counts_p, W)
    p_flat = p_tab.reshape(kk * mp, ch)

    up_p = pl.kernel(
        functools.partial(_up_body, nrows_pw, nt, mp),
        out_type=jax.ShapeDtypeStruct((NC, np_, ch), jnp.bfloat16),
        mesh=mesh,
        scratch_types=[
            pltpu.MemorySpace.VMEM_SHARED((np_, ch), jnp.bfloat16),
            pltpu.VMEM((2, RPC_C, IDXW), i32),
            pltpu.VMEM((2, RPC_C, IDXW), i32),
            pltpu.VMEM((2, RPC_C, IDXW), i32),
            pltpu.VMEM((2, RPC_C, IDXW), i32),
            pltpu.VMEM((2, RPC_C * IDXW, ch), jnp.bfloat16),
        ] + dma_sems,
        compiler_params=sc_params,
        name="sc_upsample_scatter",
    )(usrc2, ukern2, udst2, p_flat)

    bn = 4000
    out = pl.pallas_call(
        _final_body,
        grid=(n // bn,),
        in_specs=[
            pl.BlockSpec((bn, ch), lambda j: (j, 0)),
            pl.BlockSpec((NC, bn, ch), lambda j: (0, j, 0)),
            pl.BlockSpec((1, ch), lambda j: (0, 0)),
        ],
        out_specs=pl.BlockSpec((bn, ch), lambda j: (j, 0)),
        out_shape=jax.ShapeDtypeStruct((n, ch), jnp.float32),
    )(x, up_p, b.reshape(1, ch))
    return out


# speedup vs baseline: 1.0671x; 1.0671x over previous
import functools

import jax
import jax.numpy as jnp
from jax import lax
from jax.experimental import pallas as pl
from jax.experimental.pallas import tpu as pltpu
from jax.experimental.pallas import tpu_sc as plsc

M_SEG = 25000
NC = 2
NS = 16
NW = NC * NS
LANES = 16
IDXW = 128
RPC_A = 8
RPC_C = 4


def _pad_to(n, m):
    return ((n + m - 1) // m) * m


def _zero_shared(rows2, shared, tile_rows, tile_base, chunk):
    zb = jnp.zeros((2 * LANES,), shared.dtype)

    def zr(i, carry):
        rows2[0, i, :] = zb
        return carry

    lax.fori_loop(0, chunk, zr, 0)
    off = 0
    while off < tile_rows:
        sz = min(chunk, tile_rows - off)
        pltpu.sync_copy(rows2.at[0, pl.ds(0, sz), :],
                        shared.at[pl.ds(tile_base + off, sz), :])
        off += sz


def _pool_body(nrows_pw, mt, src2, dst2, x_hbm, sums_out, counts_out,
               sums_sh, srcbuf2, dstbuf2, rows2, counts,
               isem0, isem1, gsem0, gsem1, ssem0, ssem1):
    c = lax.axis_index("c")
    s = lax.axis_index("s")
    w = c * NS + s
    mp = counts.shape[0]
    rpc = RPC_A
    chunk = rpc * IDXW
    nchunk = nrows_pw // rpc
    isem = (isem0, isem1)
    gsem = (gsem0, gsem1)
    ssem = (ssem0, ssem1)
    z16 = jnp.zeros((LANES,), jnp.float32)
    ones = jnp.full((LANES,), 1.0, jnp.float32)

    def zero_counts(i, carry):
        counts[pl.ds(i * LANES, LANES)] = z16
        return carry

    lax.fori_loop(0, mp // LANES, zero_counts, 0)
    _zero_shared(rows2, sums_sh, mt, s * mt, chunk)

    def issue_idx(j, b):
        rb = w * nrows_pw + j * rpc
        pltpu.async_copy(src2.at[pl.ds(rb, rpc), :], srcbuf2.at[b], isem[b])
        pltpu.async_copy(dst2.at[pl.ds(rb, rpc), :], dstbuf2.at[b], isem[b])

    issue_idx(0, 0)
    plsc.subcore_barrier()

    def half(jj, b):
        j = jj * 2 + b
        nb = 1 - b
        rb = w * nrows_pw + j * rpc
        pltpu.make_async_copy(src2.at[pl.ds(rb, rpc), :], srcbuf2.at[b],
                              isem[b]).wait()
        pltpu.make_async_copy(dst2.at[pl.ds(rb, rpc), :], dstbuf2.at[b],
                              isem[b]).wait()
        for i in range(rpc):
            for l in range(IDXW // LANES):
                dv = dstbuf2[b, i, pl.ds(l * LANES, LANES)]
                plsc.addupdate_scatter(counts, [dv], ones)
        for i in range(rpc):
            pltpu.async_copy(x_hbm.at[srcbuf2.at[b, i]],
                             rows2.at[b, pl.ds(i * IDXW, IDXW), :], gsem[b])
        @pl.when(j >= 1)
        def _():
            pltpu.make_async_copy(x_hbm.at[pl.ds(0, chunk), :], rows2.at[nb],
                                  ssem[nb]).wait()
        @pl.when(j < nchunk - 1)
        def _():
            issue_idx(j + 1, nb)
        pltpu.make_async_copy(x_hbm.at[pl.ds(0, chunk), :], rows2.at[b],
                              gsem[b]).wait()
        for i in range(rpc):
            pltpu.async_copy(rows2.at[b, pl.ds(i * IDXW, IDXW), :],
                             sums_sh.at[dstbuf2.at[b, i]], ssem[b], add=True)

    def outer(jj, carry):
        half(jj, 0)
        half(jj, 1)
        return carry

    lax.fori_loop(0, nchunk // 2, outer, 0)
    pltpu.make_async_copy(x_hbm.at[pl.ds(0, chunk), :], rows2.at[1],
                          ssem[1]).wait()
    plsc.subcore_barrier()
    pltpu.sync_copy(sums_sh.at[pl.ds(s * mt, mt), :],
                    sums_out.at[c, pl.ds(s * mt, mt), :])
    pltpu.sync_copy(counts, counts_out.at[w])


def _up_body(nrows_pw, nt, mp, src2, kern2, dst2, p_hbm, up_out,
             up_sh, sbuf2, kbuf2, dbuf2, gbuf2, rows2,
             isem0, isem1, gsem0, gsem1, ssem0, ssem1):
    c = lax.axis_index("c")
    s = lax.axis_index("s")
    w = c * NS + s
    rpc = RPC_C
    chunk = rpc * IDXW
    nchunk = nrows_pw // rpc
    isem = (isem0, isem1)
    gsem = (gsem0, gsem1)
    ssem = (ssem0, ssem1)

    _zero_shared(rows2, up_sh, nt, s * nt, chunk)

    def issue_idx(j, b):
        rb = w * nrows_pw + j * rpc
        pltpu.async_copy(src2.at[pl.ds(rb, rpc), :], sbuf2.at[b], isem[b])
        pltpu.async_copy(kern2.at[pl.ds(rb, rpc), :], kbuf2.at[b], isem[b])
        pltpu.async_copy(dst2.at[pl.ds(rb, rpc), :], dbuf2.at[b], isem[b])

    issue_idx(0, 0)
    plsc.subcore_barrier()

    def half(jj, b):
        j = jj * 2 + b
        nb = 1 - b
        rb = w * nrows_pw + j * rpc
        pltpu.make_async_copy(src2.at[pl.ds(rb, rpc), :], sbuf2.at[b],
                              isem[b]).wait()
        pltpu.make_async_copy(kern2.at[pl.ds(rb, rpc), :], kbuf2.at[b],
                              isem[b]).wait()
        pltpu.make_async_copy(dst2.at[pl.ds(rb, rpc), :], dbuf2.at[b],
                              isem[b]).wait()
        for i in range(rpc):
            for l in range(IDXW // LANES):
                sv = sbuf2[b, i, pl.ds(l * LANES, LANES)]
                kv = kbuf2[b, i, pl.ds(l * LANES, LANES)]
                gbuf2[b, i, pl.ds(l * LANES, LANES)] = kv * mp + sv
        for i in range(rpc):
            pltpu.async_copy(p_hbm.at[gbuf2.at[b, i]],
                             rows2.at[b, pl.ds(i * IDXW, IDXW), :], gsem[b])

        @pl.when(j >= 1)
        def _():
            pltpu.make_async_copy(p_hbm.at[pl.ds(0, chunk), :], rows2.at[nb],
                                  ssem[nb]).wait()

        @pl.when(j < nchunk - 1)
        def _():
            issue_idx(j + 1, nb)

        pltpu.make_async_copy(p_hbm.at[pl.ds(0, chunk), :], rows2.at[b],
                              gsem[b]).wait()
        for i in range(rpc):
            pltpu.async_copy(rows2.at[b, pl.ds(i * IDXW, IDXW), :],
                             up_sh.at[dbuf2.at[b, i]], ssem[b], add=True)

    def outer(jj, carry):
        half(jj, 0)
        half(jj, 1)
        return carry

    lax.fori_loop(0, nchunk // 2, outer, 0)
    pltpu.make_async_copy(p_hbm.at[pl.ds(0, chunk), :], rows2.at[1],
                          ssem[1]).wait()
    plsc.subcore_barrier()
    base = s * nt
    off = 0
    while off < nt:
        sz = min(2048, nt - off)
        pltpu.sync_copy(up_sh.at[pl.ds(base + off, sz), :],
                        up_out.at[c, pl.ds(base + off, sz), :])
        off += sz


def _dense_body(sums_ref, counts_ref, w_ref, out_ref):
    sums = sums_ref[0].astype(jnp.float32) + sums_ref[1].astype(jnp.float32)
    cnt = jnp.sum(counts_ref[...], axis=0)
    pooled = sums / jnp.maximum(cnt, 1.0)[:, None]
    for k in range(out_ref.shape[0]):
        out_ref[k] = jnp.dot(pooled, w_ref[k],
                             preferred_element_type=jnp.float32
                             ).astype(jnp.bfloat16)


def _final_body(x_ref, up_ref, b_ref, o_ref):
    up = up_ref[0].astype(jnp.float32) + up_ref[1].astype(jnp.float32)
    o_ref[...] = x_ref[...] - up - b_ref[...]


def kernel(x, pool_src, pool_dst, up_src, up_dst, up_kernel, W, b):
    n, ch = x.shape
    e = pool_src.shape[0]
    kk = W.shape[0]
    i32 = jnp.int32

    mp = _pad_to(M_SEG + 1, 8 * NS)
    np_ = _pad_to(n + 1, 8 * NS)
    mt = mp // NS
    nt = np_ // NS
    epad = _pad_to(e, NW * 16 * IDXW)
    nrows_pw = epad // (NW * IDXW)
    pad = epad - e

    x_bf = x.astype(jnp.bfloat16)
    cyc = jnp.arange(pad, dtype=i32)
    ps = jnp.concatenate([pool_src.astype(i32), jnp.zeros((pad,), i32)])
    pd = jnp.concatenate([pool_dst.astype(i32), M_SEG + cyc % (mp - M_SEG)])
    us = jnp.concatenate([up_src.astype(i32), jnp.zeros((pad,), i32)])
    uk = jnp.concatenate([up_kernel.astype(i32), jnp.zeros((pad,), i32)])
    ud = jnp.concatenate([up_dst.astype(i32), n + cyc % (np_ - n)])
    src2 = ps.reshape(-1, IDXW)
    dst2 = pd.reshape(-1, IDXW)
    usrc2 = us.reshape(-1, IDXW)
    ukern2 = uk.reshape(-1, IDXW)
    udst2 = ud.reshape(-1, IDXW)

    mesh = plsc.VectorSubcoreMesh(core_axis_name="c", subcore_axis_name="s")
    sc_params = pltpu.CompilerParams(needs_layout_passes=False,
                                     use_tc_tiling_on_sc=False)
    dma_sems = [pltpu.SemaphoreType.DMA] * 6

    sums_p, counts_p = pl.kernel(
        functools.partial(_pool_body, nrows_pw, mt),
        out_type=(jax.ShapeDtypeStruct((NC, mp, ch), jnp.bfloat16),
                  jax.ShapeDtypeStruct((NW, mp), jnp.float32)),
        mesh=mesh,
        scratch_types=[
            pltpu.MemorySpace.VMEM_SHARED((mp, ch), jnp.bfloat16),
            pltpu.VMEM((2, RPC_A, IDXW), i32),
            pltpu.VMEM((2, RPC_A, IDXW), i32),
            pltpu.VMEM((2, RPC_A * IDXW, ch), jnp.bfloat16),
            pltpu.VMEM((mp,), jnp.float32),
        ] + dma_sems,
        compiler_params=sc_params,
        name="sc_pool_segment_sum",
    )(src2, dst2, x_bf)

    bm = mp // 14
    p_tab = pl.pallas_call(
        _dense_body,
        grid=(mp // bm,),
        in_specs=[
            pl.BlockSpec((NC, bm, ch), lambda j: (0, j, 0)),
            pl.BlockSpec((NW, bm), lambda j: (0, j)),
            pl.BlockSpec((kk, ch, ch), lambda j: (0, 0, 0)),
        ],
        out_specs=pl.BlockSpec((kk, bm, ch), lambda j: (0, j, 0)),
        out_shape=jax.ShapeDtypeStruct((kk, mp, ch), jnp.bfloat16),
    )(sums_p, counts_p, W)
    p_flat = p_tab.reshape(kk * mp, ch)

    up_p = pl.kernel(
        functools.partial(_up_body, nrows_pw, nt, mp),
        out_type=jax.ShapeDtypeStruct((NC, np_, ch), jnp.bfloat16),
        mesh=mesh,
        scratch_types=[
            pltpu.MemorySpace.VMEM_SHARED((np_, ch), jnp.bfloat16),
            pltpu.VMEM((2, RPC_C, IDXW), i32),
            pltpu.VMEM((2, RPC_C, IDXW), i32),
            pltpu.VMEM((2, RPC_C, IDXW), i32),
            pltpu.VMEM((2, RPC_C, IDXW), i32),
            pltpu.VMEM((2, RPC_C * IDXW, ch), jnp.bfloat16),
        ] + dma_sems,
        compiler_params=sc_params,
        name="sc_upsample_scatter",
    )(usrc2, ukern2, udst2, p_flat)

    bn = 4000
    out = pl.pallas_call(
        _final_body,
        grid=(n // bn,),
        in_specs=[
            pl.BlockSpec((bn, ch), lambda j: (j, 0)),
            pl.BlockSpec((NC, bn, ch), lambda j: (0, j, 0)),
            pl.BlockSpec((1, ch), lambda j: (0, 0)),
        ],
        out_specs=pl.BlockSpec((bn, ch), lambda j: (j, 0)),
        out_shape=jax.ShapeDtypeStruct((n, ch), jnp.float32),
    )(x, up_p, b.reshape(1, ch))
    return out


# speedup vs baseline: 1.1490x; 1.0767x over previous
import functools

import jax
import jax.numpy as jnp
from jax import lax
from jax.experimental import pallas as pl
from jax.experimental.pallas import tpu as pltpu
from jax.experimental.pallas import tpu_sc as plsc

M_SEG = 25000
NC = 2
NS = 16
NW = NC * NS
LANES = 16
IDXW = 128
RPC_A = 8
RPC_C = 4


def _pad_to(n, m):
    return ((n + m - 1) // m) * m


def _zero_shared(rows2, shared, tile_rows, tile_base, chunk):
    zb = jnp.zeros((2 * LANES,), shared.dtype)

    def zr(i, carry):
        rows2[0, i, :] = zb
        return carry

    lax.fori_loop(0, chunk, zr, 0)
    off = 0
    while off < tile_rows:
        sz = min(chunk, tile_rows - off)
        pltpu.sync_copy(rows2.at[0, pl.ds(0, sz), :],
                        shared.at[pl.ds(tile_base + off, sz), :])
        off += sz


def _pool_body(nrows_pw, mt, src2, dst2, x_hbm, sums_out, counts_out,
               sums_sh, srcbuf2, dstbuf2, rows2, counts,
               isem0, isem1, gsem0, gsem1, ssem0, ssem1):
    c = lax.axis_index("c")
    s = lax.axis_index("s")
    w = c * NS + s
    mp = counts.shape[0]
    rpc = RPC_A
    chunk = rpc * IDXW
    nchunk = nrows_pw // rpc
    isem = (isem0, isem1)
    gsem = (gsem0, gsem1)
    ssem = (ssem0, ssem1)
    z16 = jnp.zeros((LANES,), jnp.float32)
    ones = jnp.full((LANES,), 1.0, jnp.float32)

    def zero_counts(i, carry):
        counts[pl.ds(i * LANES, LANES)] = z16
        return carry

    lax.fori_loop(0, mp // LANES, zero_counts, 0)
    _zero_shared(rows2, sums_sh, mt, s * mt, chunk)

    def issue_idx(j, b):
        rb = w * nrows_pw + j * rpc
        pltpu.async_copy(src2.at[pl.ds(rb, rpc), :], srcbuf2.at[b], isem[b])
        pltpu.async_copy(dst2.at[pl.ds(rb, rpc), :], dstbuf2.at[b], isem[b])

    issue_idx(0, 0)
    plsc.subcore_barrier()

    def half(jj, b):
        j = jj * 2 + b
        nb = 1 - b
        rb = w * nrows_pw + j * rpc
        pltpu.make_async_copy(src2.at[pl.ds(rb, rpc), :], srcbuf2.at[b],
                              isem[b]).wait()
        pltpu.make_async_copy(dst2.at[pl.ds(rb, rpc), :], dstbuf2.at[b],
                              isem[b]).wait()
        for i in range(rpc):
            for l in range(IDXW // LANES):
                dv = dstbuf2[b, i, pl.ds(l * LANES, LANES)]
                plsc.addupdate_scatter(counts, [dv], ones)
        for i in range(rpc):
            pltpu.async_copy(x_hbm.at[srcbuf2.at[b, i]],
                             rows2.at[b, pl.ds(i * IDXW, IDXW), :], gsem[b])
        @pl.when(j >= 1)
        def _():
            pltpu.make_async_copy(x_hbm.at[pl.ds(0, chunk), :], rows2.at[nb],
                                  ssem[nb]).wait()
        @pl.when(j < nchunk - 1)
        def _():
            issue_idx(j + 1, nb)
        pltpu.make_async_copy(x_hbm.at[pl.ds(0, chunk), :], rows2.at[b],
                              gsem[b]).wait()
        for i in range(rpc):
            pltpu.async_copy(rows2.at[b, pl.ds(i * IDXW, IDXW), :],
                             sums_sh.at[dstbuf2.at[b, i]], ssem[b], add=True)

    def outer(jj, carry):
        half(jj, 0)
        half(jj, 1)
        return carry

    lax.fori_loop(0, nchunk // 2, outer, 0)
    pltpu.make_async_copy(x_hbm.at[pl.ds(0, chunk), :], rows2.at[1],
                          ssem[1]).wait()
    plsc.subcore_barrier()
    pltpu.sync_copy(sums_sh.at[pl.ds(s * mt, mt), :],
                    sums_out.at[c, pl.ds(s * mt, mt), :])
    pltpu.sync_copy(counts, counts_out.at[w])


def _up_body(nrows_pw, nt, mp, src2, kern2, dst2, p_hbm, up_out,
             up_sh, sbuf2, kbuf2, dbuf2, gbuf2, rows2,
             isem0, isem1, gsem0, gsem1, ssem0, ssem1):
    c = lax.axis_index("c")
    s = lax.axis_index("s")
    w = c * NS + s
    rpc = RPC_C
    chunk = rpc * IDXW
    nchunk = nrows_pw // rpc
    isem = (isem0, isem1)
    gsem = (gsem0, gsem1)
    ssem = (ssem0, ssem1)

    _zero_shared(rows2, up_sh, nt, s * nt, chunk)

    def issue_idx(j, b):
        rb = w * nrows_pw + j * rpc
        pltpu.async_copy(src2.at[pl.ds(rb, rpc), :], sbuf2.at[b], isem[b])
        pltpu.async_copy(kern2.at[pl.ds(rb, rpc), :], kbuf2.at[b], isem[b])
        pltpu.async_copy(dst2.at[pl.ds(rb, rpc), :], dbuf2.at[b], isem[b])

    issue_idx(0, 0)
    plsc.subcore_barrier()

    def half(jj, b):
        j = jj * 2 + b
        nb = 1 - b
        rb = w * nrows_pw + j * rpc
        pltpu.make_async_copy(src2.at[pl.ds(rb, rpc), :], sbuf2.at[b],
                              isem[b]).wait()
        pltpu.make_async_copy(kern2.at[pl.ds(rb, rpc), :], kbuf2.at[b],
                              isem[b]).wait()
        pltpu.make_async_copy(dst2.at[pl.ds(rb, rpc), :], dbuf2.at[b],
                              isem[b]).wait()
        for i in range(rpc):
            for l in range(IDXW // LANES):
                sv = sbuf2[b, i, pl.ds(l * LANES, LANES)]
                kv = kbuf2[b, i, pl.ds(l * LANES, LANES)]
                gbuf2[b, i, pl.ds(l * LANES, LANES)] = kv * mp + sv
        for i in range(rpc):
            pltpu.async_copy(p_hbm.at[gbuf2.at[b, i]],
                             rows2.at[b, pl.ds(i * IDXW, IDXW), :], gsem[b])

        @pl.when(j >= 1)
        def _():
            pltpu.make_async_copy(p_hbm.at[pl.ds(0, chunk), :], rows2.at[nb],
                                  ssem[nb]).wait()

        @pl.when(j < nchunk - 1)
        def _():
            issue_idx(j + 1, nb)

        pltpu.make_async_copy(p_hbm.at[pl.ds(0, chunk), :], rows2.at[b],
                              gsem[b]).wait()
        for i in range(rpc):
            pltpu.async_copy(rows2.at[b, pl.ds(i * IDXW, IDXW), :],
                             up_sh.at[dbuf2.at[b, i]], ssem[b], add=True)

    def outer(jj, carry):
        half(jj, 0)
        half(jj, 1)
        return carry

    lax.fori_loop(0, nchunk // 2, outer, 0)
    pltpu.make_async_copy(p_hbm.at[pl.ds(0, chunk), :], rows2.at[1],
                          ssem[1]).wait()
    plsc.subcore_barrier()
    base = s * nt
    off = 0
    while off < nt:
        sz = min(2048, nt - off)
        pltpu.sync_copy(up_sh.at[pl.ds(base + off, sz), :],
                        up_out.at[c, pl.ds(base + off, sz), :])
        off += sz


def _dense_body(sums_ref, counts_ref, w_ref, out_ref):
    sums = sums_ref[0].astype(jnp.float32) + sums_ref[1].astype(jnp.float32)
    cnt = jnp.sum(counts_ref[...], axis=0)
    pooled = sums / jnp.maximum(cnt, 1.0)[:, None]
    out_ref[...] = jnp.dot(pooled, w_ref[0],
                           preferred_element_type=jnp.float32
                           ).astype(jnp.bfloat16)


def _final_body(x_ref, up_ref, b_ref, o_ref):
    up = up_ref[0].astype(jnp.float32) + up_ref[1].astype(jnp.float32)
    o_ref[...] = x_ref[...] - up - b_ref[...]


def kernel(x, pool_src, pool_dst, up_src, up_dst, up_kernel, W, b):
    n, ch = x.shape
    e = pool_src.shape[0]
    kk = W.shape[0]
    i32 = jnp.int32

    mp = _pad_to(M_SEG + 1, 8 * NS)
    np_ = _pad_to(n + 1, 8 * NS)
    mt = mp // NS
    nt = np_ // NS
    epad = _pad_to(e, NW * 16 * IDXW)
    nrows_pw = epad // (NW * IDXW)
    pad = epad - e

    x_bf = x.astype(jnp.bfloat16)
    cyc = jnp.arange(pad, dtype=i32)
    ps = jnp.concatenate([pool_src.astype(i32), cyc % n])
    pd = jnp.concatenate([pool_dst.astype(i32), M_SEG + cyc % (mp - M_SEG)])
    us = jnp.concatenate([up_src.astype(i32), cyc % M_SEG])
    uk = jnp.concatenate([up_kernel.astype(i32), jnp.zeros((pad,), i32)])
    ud = jnp.concatenate([up_dst.astype(i32), n + cyc % (np_ - n)])
    src2 = ps.reshape(-1, IDXW)
    dst2 = pd.reshape(-1, IDXW)
    usrc2 = us.reshape(-1, IDXW)
    ukern2 = uk.reshape(-1, IDXW)
    udst2 = ud.reshape(-1, IDXW)

    mesh = plsc.VectorSubcoreMesh(core_axis_name="c", subcore_axis_name="s")
    sc_params = pltpu.CompilerParams(needs_layout_passes=False,
                                     use_tc_tiling_on_sc=False)
    dma_sems = [pltpu.SemaphoreType.DMA] * 6

    sums_p, counts_p = pl.kernel(
        functools.partial(_pool_body, nrows_pw, mt),
        out_type=(jax.ShapeDtypeStruct((NC, mp, ch), jnp.bfloat16),
                  jax.ShapeDtypeStruct((NW, mp), jnp.float32)),
        mesh=mesh,
        scratch_types=[
            pltpu.MemorySpace.VMEM_SHARED((mp, ch), jnp.bfloat16),
            pltpu.VMEM((2, RPC_A, IDXW), i32),
            pltpu.VMEM((2, RPC_A, IDXW), i32),
            pltpu.VMEM((2, RPC_A * IDXW, ch), jnp.bfloat16),
            pltpu.VMEM((mp,), jnp.float32),
        ] + dma_sems,
        compiler_params=sc_params,
        name="sc_pool_segment_sum",
    )(src2, dst2, x_bf)

    bm = mp // 14
    mb = mp // bm
    p_flat = pl.pallas_call(
        _dense_body,
        grid=(kk, mb),
        in_specs=[
            pl.BlockSpec((NC, bm, ch), lambda k, j: (0, j, 0)),
            pl.BlockSpec((NW, bm), lambda k, j: (0, j)),
            pl.BlockSpec((1, ch, ch), lambda k, j: (k, 0, 0)),
        ],
        out_specs=pl.BlockSpec((bm, ch), lambda k, j: (k * mb + j, 0)),
        out_shape=jax.ShapeDtypeStruct((kk * mp, ch), jnp.bfloat16),
    )(sums_p, counts_p, W)

    up_p = pl.kernel(
        functools.partial(_up_body, nrows_pw, nt, mp),
        out_type=jax.ShapeDtypeStruct((NC, np_, ch), jnp.bfloat16),
        mesh=mesh,
        scratch_types=[
            pltpu.MemorySpace.VMEM_SHARED((np_, ch), jnp.bfloat16),
            pltpu.VMEM((2, RPC_C, IDXW), i32),
            pltpu.VMEM((2, RPC_C, IDXW), i32),
            pltpu.VMEM((2, RPC_C, IDXW), i32),
            pltpu.VMEM((2, RPC_C, IDXW), i32),
            pltpu.VMEM((2, RPC_C * IDXW, ch), jnp.bfloat16),
        ] + dma_sems,
        compiler_params=sc_params,
        name="sc_upsample_scatter",
    )(usrc2, ukern2, udst2, p_flat)

    bn = 4000
    out = pl.pallas_call(
        _final_body,
        grid=(n // bn,),
        in_specs=[
            pl.BlockSpec((bn, ch), lambda j: (j, 0)),
            pl.BlockSpec((NC, bn, ch), lambda j: (0, j, 0)),
            pl.BlockSpec((1, ch), lambda j: (0, 0)),
        ],
        out_specs=pl.BlockSpec((bn, ch), lambda j: (j, 0)),
        out_shape=jax.ShapeDtypeStruct((n, ch), jnp.float32),
    )(x, up_p, b.reshape(1, ch))
    return out


# speedup vs baseline: 1.2514x; 1.0891x over previous
import functools

import jax
import jax.numpy as jnp
from jax import lax
from jax.experimental import pallas as pl
from jax.experimental.pallas import tpu as pltpu
from jax.experimental.pallas import tpu_sc as plsc

M_SEG = 25000
NC = 2
NS = 16
NW = NC * NS
LANES = 16
IDXW = 128
RPC_A = 8
RPC_C = 4


def _pad_to(n, m):
    return ((n + m - 1) // m) * m


def _zero_shared(rows2, shared, tile_rows, tile_base, chunk):
    zb = jnp.zeros((2 * LANES,), shared.dtype)

    def zr(i, carry):
        rows2[0, i, :] = zb
        return carry

    lax.fori_loop(0, chunk, zr, 0)
    off = 0
    while off < tile_rows:
        sz = min(chunk, tile_rows - off)
        pltpu.sync_copy(rows2.at[0, pl.ds(0, sz), :],
                        shared.at[pl.ds(tile_base + off, sz), :])
        off += sz


def _pool_body(nrows_pw, mt, src2, dst2, x_hbm, sums_out, counts_out,
               sums_sh, srcbuf2, dstbuf2, rows2, counts,
               isem0, isem1, gsem0, gsem1, ssem0, ssem1):
    c = lax.axis_index("c")
    s = lax.axis_index("s")
    w = c * NS + s
    mp = counts.shape[0]
    rpc = RPC_A
    chunk = rpc * IDXW
    nchunk = nrows_pw // rpc
    isem = (isem0, isem1)
    gsem = (gsem0, gsem1)
    ssem = (ssem0, ssem1)
    z16 = jnp.zeros((LANES,), jnp.float32)
    ones = jnp.full((LANES,), 1.0, jnp.float32)

    def zero_counts(i, carry):
        counts[pl.ds(i * LANES, LANES)] = z16
        return carry

    lax.fori_loop(0, mp // LANES, zero_counts, 0)
    _zero_shared(rows2, sums_sh, mt, s * mt, chunk)

    def issue_idx(j, b):
        rb = w * nrows_pw + j * rpc
        pltpu.async_copy(src2.at[pl.ds(rb, rpc), :], srcbuf2.at[b], isem[b])
        pltpu.async_copy(dst2.at[pl.ds(rb, rpc), :], dstbuf2.at[b], isem[b])

    issue_idx(0, 0)
    plsc.subcore_barrier()

    def half(jj, b):
        j = jj * 2 + b
        nb = 1 - b
        rb = w * nrows_pw + j * rpc
        pltpu.make_async_copy(src2.at[pl.ds(rb, rpc), :], srcbuf2.at[b],
                              isem[b]).wait()
        pltpu.make_async_copy(dst2.at[pl.ds(rb, rpc), :], dstbuf2.at[b],
                              isem[b]).wait()
        for i in range(rpc):
            for l in range(IDXW // LANES):
                dv = dstbuf2[b, i, pl.ds(l * LANES, LANES)]
                plsc.addupdate_scatter(counts, [dv], ones)
        for i in range(rpc):
            pltpu.async_copy(x_hbm.at[srcbuf2.at[b, i]],
                             rows2.at[b, pl.ds(i * IDXW, IDXW), :], gsem[b])
        @pl.when(j >= 1)
        def _():
            pltpu.make_async_copy(x_hbm.at[pl.ds(0, chunk), :], rows2.at[nb],
                                  ssem[nb]).wait()
        @pl.when(j < nchunk - 1)
        def _():
            issue_idx(j + 1, nb)
        pltpu.make_async_copy(x_hbm.at[pl.ds(0, chunk), :], rows2.at[b],
                              gsem[b]).wait()
        for i in range(rpc):
            pltpu.async_copy(rows2.at[b, pl.ds(i * IDXW, IDXW), :],
                             sums_sh.at[dstbuf2.at[b, i]], ssem[b], add=True)

    def outer(jj, carry):
        half(jj, 0)
        half(jj, 1)
        return carry

    lax.fori_loop(0, nchunk // 2, outer, 0)
    pltpu.make_async_copy(x_hbm.at[pl.ds(0, chunk), :], rows2.at[1],
                          ssem[1]).wait()
    plsc.subcore_barrier()
    pltpu.sync_copy(sums_sh.at[pl.ds(s * mt, mt), :],
                    sums_out.at[c, pl.ds(s * mt, mt), :])
    pltpu.sync_copy(counts, counts_out.at[w])


def _up_body(nrows_pw, nt, mp, src2, kern2, dst2, p_hbm, up_out,
             up_sh, sbuf2, kbuf2, dbuf2, gbuf2, rows2,
             isem0, isem1, gsem0, gsem1, ssem0, ssem1):
    c = lax.axis_index("c")
    s = lax.axis_index("s")
    w = c * NS + s
    rpc = RPC_C
    chunk = rpc * IDXW
    nchunk = nrows_pw // rpc
    isem = (isem0, isem1)
    gsem = (gsem0, gsem1)
    ssem = (ssem0, ssem1)

    _zero_shared(rows2, up_sh, nt, s * nt, chunk)

    def issue_idx(j, b):
        rb = w * nrows_pw + j * rpc
        pltpu.async_copy(src2.at[pl.ds(rb, rpc), :], sbuf2.at[b], isem[b])
        pltpu.async_copy(kern2.at[pl.ds(rb, rpc), :], kbuf2.at[b], isem[b])
        pltpu.async_copy(dst2.at[pl.ds(rb, rpc), :], dbuf2.at[b], isem[b])

    issue_idx(0, 0)
    plsc.subcore_barrier()

    def half(jj, b):
        j = jj * 2 + b
        nb = 1 - b
        rb = w * nrows_pw + j * rpc
        pltpu.make_async_copy(src2.at[pl.ds(rb, rpc), :], sbuf2.at[b],
                              isem[b]).wait()
        pltpu.make_async_copy(kern2.at[pl.ds(rb, rpc), :], kbuf2.at[b],
                              isem[b]).wait()
        pltpu.make_async_copy(dst2.at[pl.ds(rb, rpc), :], dbuf2.at[b],
                              isem[b]).wait()
        for i in range(rpc):
            for l in range(IDXW // LANES):
                sv = sbuf2[b, i, pl.ds(l * LANES, LANES)]
                kv = kbuf2[b, i, pl.ds(l * LANES, LANES)]
                gbuf2[b, i, pl.ds(l * LANES, LANES)] = kv * mp + sv
        for i in range(rpc):
            pltpu.async_copy(p_hbm.at[gbuf2.at[b, i]],
                             rows2.at[b, pl.ds(i * IDXW, IDXW), :], gsem[b])

        @pl.when(j >= 1)
        def _():
            pltpu.make_async_copy(p_hbm.at[pl.ds(0, chunk), :], rows2.at[nb],
                                  ssem[nb]).wait()

        @pl.when(j < nchunk - 1)
        def _():
            issue_idx(j + 1, nb)

        pltpu.make_async_copy(p_hbm.at[pl.ds(0, chunk), :], rows2.at[b],
                              gsem[b]).wait()
        for i in range(rpc):
            pltpu.async_copy(rows2.at[b, pl.ds(i * IDXW, IDXW), :],
                             up_sh.at[dbuf2.at[b, i]], ssem[b], add=True)

    def outer(jj, carry):
        half(jj, 0)
        half(jj, 1)
        return carry

    lax.fori_loop(0, nchunk // 2, outer, 0)
    pltpu.make_async_copy(p_hbm.at[pl.ds(0, chunk), :], rows2.at[1],
                          ssem[1]).wait()
    plsc.subcore_barrier()
    base = s * nt
    off = 0
    while off < nt:
        sz = min(2048, nt - off)
        pltpu.sync_copy(up_sh.at[pl.ds(base + off, sz), :],
                        up_out.at[c, pl.ds(base + off, sz), :])
        off += sz


def _dense_body(sums_ref, counts_ref, w_ref, out_ref):
    sums = sums_ref[0].astype(jnp.float32) + sums_ref[1].astype(jnp.float32)
    cnt = jnp.sum(counts_ref[...], axis=0)
    pooled = sums / jnp.maximum(cnt, 1.0)[:, None]
    out_ref[...] = jnp.dot(pooled, w_ref[0],
                           preferred_element_type=jnp.float32
                           ).astype(jnp.bfloat16)


def _final_body(x_ref, up_ref, b_ref, o_ref):
    up = up_ref[0].astype(jnp.float32) + up_ref[1].astype(jnp.float32)
    o_ref[...] = x_ref[...] - up - b_ref[...]


def kernel(x, pool_src, pool_dst, up_src, up_dst, up_kernel, W, b):
    n, ch = x.shape
    e = pool_src.shape[0]
    kk = W.shape[0]
    i32 = jnp.int32

    mp = _pad_to(M_SEG + 1, 8 * NS)
    np_ = _pad_to(n + 1, 8 * NS)
    mt = mp // NS
    nt = np_ // NS
    epad = _pad_to(e, NW * 16 * IDXW)
    nrows_pw = epad // (NW * IDXW)
    pad = epad - e

    x_bf = x.astype(jnp.bfloat16)
    cyc = jnp.arange(pad, dtype=i32)
    ps = jnp.concatenate([pool_src.astype(i32), cyc % n])
    pd = jnp.concatenate([pool_dst.astype(i32), M_SEG + cyc % (mp - M_SEG)])
    us = jnp.concatenate([up_src.astype(i32), cyc % M_SEG])
    uk = jnp.concatenate([up_kernel.astype(i32), jnp.zeros((pad,), i32)])
    ud = jnp.concatenate([up_dst.astype(i32), n + cyc % (np_ - n)])
    src2 = ps.reshape(-1, IDXW)
    dst2 = pd.reshape(-1, IDXW)
    usrc2 = us.reshape(-1, IDXW)
    ukern2 = uk.reshape(-1, IDXW)
    udst2 = ud.reshape(-1, IDXW)

    mesh = plsc.VectorSubcoreMesh(core_axis_name="c", subcore_axis_name="s")
    sc_params = pltpu.CompilerParams(needs_layout_passes=False,
                                     use_tc_tiling_on_sc=False)
    dma_sems = [pltpu.SemaphoreType.DMA] * 6

    sums_p, counts_p = pl.kernel(
        functools.partial(_pool_body, nrows_pw, mt),
        out_type=(jax.ShapeDtypeStruct((NC, mp, ch), jnp.bfloat16),
                  jax.ShapeDtypeStruct((NW, mp), jnp.float32)),
        mesh=mesh,
        scratch_types=[
            pltpu.MemorySpace.VMEM_SHARED((mp, ch), jnp.bfloat16),
            pltpu.VMEM((2, RPC_A, IDXW), i32),
            pltpu.VMEM((2, RPC_A, IDXW), i32),
            pltpu.VMEM((2, RPC_A * IDXW, ch), jnp.bfloat16),
            pltpu.VMEM((mp,), jnp.float32),
        ] + dma_sems,
        compiler_params=sc_params,
        name="sc_pool_segment_sum",
    )(src2, dst2, x_bf)

    bm = mp // 14
    mb = mp // bm
    p4 = pl.pallas_call(
        _dense_body,
        grid=(mb, kk),
        in_specs=[
            pl.BlockSpec((NC, bm, ch), lambda j, k: (0, j, 0)),
            pl.BlockSpec((NW, bm), lambda j, k: (0, j)),
            pl.BlockSpec((1, ch, ch), lambda j, k: (k, 0, 0)),
        ],
        out_specs=pl.BlockSpec((bm, ch), lambda j, k: (k * mb + j, 0)),
        out_shape=jax.ShapeDtypeStruct((kk * mp, ch), jnp.bfloat16),
    )(sums_p, counts_p, W)
    p_flat = p4

    up_p = pl.kernel(
        functools.partial(_up_body, nrows_pw, nt, mp),
        out_type=jax.ShapeDtypeStruct((NC, np_, ch), jnp.bfloat16),
        mesh=mesh,
        scratch_types=[
            pltpu.MemorySpace.VMEM_SHARED((np_, ch), jnp.bfloat16),
            pltpu.VMEM((2, RPC_C, IDXW), i32),
            pltpu.VMEM((2, RPC_C, IDXW), i32),
            pltpu.VMEM((2, RPC_C, IDXW), i32),
            pltpu.VMEM((2, RPC_C, IDXW), i32),
            pltpu.VMEM((2, RPC_C * IDXW, ch), jnp.bfloat16),
        ] + dma_sems,
        compiler_params=sc_params,
        name="sc_upsample_scatter",
    )(usrc2, ukern2, udst2, p_flat)

    bn = 4000
    out = pl.pallas_call(
        _final_body,
        grid=(n // bn,),
        in_specs=[
            pl.BlockSpec((bn, ch), lambda j: (j, 0)),
            pl.BlockSpec((NC, bn, ch), lambda j: (0, j, 0)),
            pl.BlockSpec((1, ch), lambda j: (0, 0)),
        ],
        out_specs=pl.BlockSpec((bn, ch), lambda j: (j, 0)),
        out_shape=jax.ShapeDtypeStruct((n, ch), jnp.float32),
    )(x, up_p, b.reshape(1, ch))
    return out


# speedup vs baseline: 1.5370x; 1.2282x over previous
import functools

import jax
import jax.numpy as jnp
from jax import lax
from jax.experimental import pallas as pl
from jax.experimental.pallas import tpu as pltpu
from jax.experimental.pallas import tpu_sc as plsc

M_SEG = 25000
NC = 2
NS = 16
NW = NC * NS
LANES = 16
IDXW = 128
RPC_A = 8
RPC_C = 4


def _pad_to(n, m):
    return ((n + m - 1) // m) * m


def _zero_shared(rows2, shared, tile_rows, tile_base, chunk):
    zb = jnp.zeros((2 * LANES,), shared.dtype)

    def zr(i, carry):
        rows2[0, i, :] = zb
        return carry

    lax.fori_loop(0, chunk, zr, 0)
    off = 0
    while off < tile_rows:
        sz = min(chunk, tile_rows - off)
        pltpu.sync_copy(rows2.at[0, pl.ds(0, sz), :],
                        shared.at[pl.ds(tile_base + off, sz), :])
        off += sz


def _pool_body(nrows_pw, mt, src2, dst2, x_hbm, sums_out, counts_out,
               sums_sh, srcbuf2, dstbuf2, rows2, counts,
               isem0, isem1, gsem0, gsem1, ssem0, ssem1):
    c = lax.axis_index("c")
    s = lax.axis_index("s")
    w = c * NS + s
    mp = counts.shape[0]
    rpc = RPC_A
    chunk = rpc * IDXW
    nchunk = nrows_pw // rpc
    isem = (isem0, isem1)
    gsem = (gsem0, gsem1)
    ssem = (ssem0, ssem1)
    z16 = jnp.zeros((LANES,), jnp.float32)
    ones = jnp.full((LANES,), 1.0, jnp.float32)

    def zero_counts(i, carry):
        counts[pl.ds(i * LANES, LANES)] = z16
        return carry

    lax.fori_loop(0, mp // LANES, zero_counts, 0)
    _zero_shared(rows2, sums_sh, mt, s * mt, chunk)

    def issue_idx(j, b):
        rb = w * nrows_pw + j * rpc
        pltpu.async_copy(src2.at[pl.ds(rb, rpc), :], srcbuf2.at[b], isem[b])
        pltpu.async_copy(dst2.at[pl.ds(rb, rpc), :], dstbuf2.at[b], isem[b])

    issue_idx(0, 0)
    plsc.subcore_barrier()

    def half(jj, b):
        j = jj * 2 + b
        nb = 1 - b
        rb = w * nrows_pw + j * rpc
        pltpu.make_async_copy(src2.at[pl.ds(rb, rpc), :], srcbuf2.at[b],
                              isem[b]).wait()
        pltpu.make_async_copy(dst2.at[pl.ds(rb, rpc), :], dstbuf2.at[b],
                              isem[b]).wait()
        for i in range(rpc):
            for l in range(IDXW // LANES):
                dv = dstbuf2[b, i, pl.ds(l * LANES, LANES)]
                plsc.addupdate_scatter(counts, [dv], ones)
        for i in range(rpc):
            pltpu.async_copy(x_hbm.at[srcbuf2.at[b, i]],
                             rows2.at[b, pl.ds(i * IDXW, IDXW), :], gsem[b])
        @pl.when(j >= 1)
        def _():
            pltpu.make_async_copy(x_hbm.at[pl.ds(0, chunk), :], rows2.at[nb],
                                  ssem[nb]).wait()
        @pl.when(j < nchunk - 1)
        def _():
            issue_idx(j + 1, nb)
        pltpu.make_async_copy(x_hbm.at[pl.ds(0, chunk), :], rows2.at[b],
                              gsem[b]).wait()
        for i in range(rpc):
            pltpu.async_copy(rows2.at[b, pl.ds(i * IDXW, IDXW), :],
                             sums_sh.at[dstbuf2.at[b, i]], ssem[b], add=True)

    def outer(jj, carry):
        half(jj, 0)
        half(jj, 1)
        return carry

    lax.fori_loop(0, nchunk // 2, outer, 0)
    pltpu.make_async_copy(x_hbm.at[pl.ds(0, chunk), :], rows2.at[1],
                          ssem[1]).wait()
    plsc.subcore_barrier()
    pltpu.sync_copy(sums_sh.at[pl.ds(s * mt, mt), :],
                    sums_out.at[c, pl.ds(s * mt, mt), :])
    pltpu.sync_copy(counts, counts_out.at[w])


def _up_body(nrows_pw, nt, mp, src2, kern2, dst2, p_hbm, up_out,
             up_sh, sbuf2, kbuf2, dbuf2, gbuf2, rows2,
             isem0, isem1, gsem0, gsem1, ssem0, ssem1):
    c = lax.axis_index("c")
    s = lax.axis_index("s")
    w = c * NS + s
    rpc = RPC_C
    chunk = rpc * IDXW
    nchunk = nrows_pw // rpc
    isem = (isem0, isem1)
    gsem = (gsem0, gsem1)
    ssem = (ssem0, ssem1)

    _zero_shared(rows2, up_sh, nt, s * nt, chunk)

    def issue_idx(j, b):
        rb = w * nrows_pw + j * rpc
        pltpu.async_copy(src2.at[pl.ds(rb, rpc), :], sbuf2.at[b], isem[b])
        pltpu.async_copy(kern2.at[pl.ds(rb, rpc), :], kbuf2.at[b], isem[b])
        pltpu.async_copy(dst2.at[pl.ds(rb, rpc), :], dbuf2.at[b], isem[b])

    issue_idx(0, 0)
    plsc.subcore_barrier()

    def half(jj, b):
        j = jj * 2 + b
        nb = 1 - b
        rb = w * nrows_pw + j * rpc
        pltpu.make_async_copy(src2.at[pl.ds(rb, rpc), :], sbuf2.at[b],
                              isem[b]).wait()
        pltpu.make_async_copy(kern2.at[pl.ds(rb, rpc), :], kbuf2.at[b],
                              isem[b]).wait()
        pltpu.make_async_copy(dst2.at[pl.ds(rb, rpc), :], dbuf2.at[b],
                              isem[b]).wait()
        for i in range(rpc):
            for l in range(IDXW // LANES):
                sv = sbuf2[b, i, pl.ds(l * LANES, LANES)]
                kv = kbuf2[b, i, pl.ds(l * LANES, LANES)]
                gbuf2[b, i, pl.ds(l * LANES, LANES)] = kv * mp + sv
        for i in range(rpc):
            pltpu.async_copy(p_hbm.at[gbuf2.at[b, i]],
                             rows2.at[b, pl.ds(i * IDXW, IDXW), :], gsem[b])

        @pl.when(j >= 1)
        def _():
            pltpu.make_async_copy(p_hbm.at[pl.ds(0, chunk), :], rows2.at[nb],
                                  ssem[nb]).wait()

        @pl.when(j < nchunk - 1)
        def _():
            issue_idx(j + 1, nb)

        pltpu.make_async_copy(p_hbm.at[pl.ds(0, chunk), :], rows2.at[b],
                              gsem[b]).wait()
        for i in range(rpc):
            pltpu.async_copy(rows2.at[b, pl.ds(i * IDXW, IDXW), :],
                             up_sh.at[dbuf2.at[b, i]], ssem[b], add=True)

    def outer(jj, carry):
        half(jj, 0)
        half(jj, 1)
        return carry

    lax.fori_loop(0, nchunk // 2, outer, 0)
    pltpu.make_async_copy(p_hbm.at[pl.ds(0, chunk), :], rows2.at[1],
                          ssem[1]).wait()
    plsc.subcore_barrier()
    base = s * nt
    off = 0
    while off < nt:
        sz = min(2048, nt - off)
        pltpu.sync_copy(up_sh.at[pl.ds(base + off, sz), :],
                        up_out.at[c, pl.ds(base + off, sz), :])
        off += sz


def _dense_body(sums_ref, counts_ref, w_ref, out_ref):
    sums = sums_ref[0].astype(jnp.float32) + sums_ref[1].astype(jnp.float32)
    cnt = jnp.sum(counts_ref[...], axis=0)
    pooled = sums / jnp.maximum(cnt, 1.0)[:, None]
    for k in range(out_ref.shape[0]):
        out_ref[k] = jnp.dot(pooled, w_ref[k],
                             preferred_element_type=jnp.float32
                             ).astype(jnp.bfloat16)


def _final_body(x_ref, up_ref, b_ref, o_ref):
    up = up_ref[0].astype(jnp.float32) + up_ref[1].astype(jnp.float32)
    o_ref[...] = x_ref[...] - up - b_ref[...]


def kernel(x, pool_src, pool_dst, up_src, up_dst, up_kernel, W, b):
    n, ch = x.shape
    e = pool_src.shape[0]
    kk = W.shape[0]
    i32 = jnp.int32

    mp = _pad_to(M_SEG + 1, 8 * NS)
    np_ = _pad_to(n + 1, 8 * NS)
    mt = mp // NS
    nt = np_ // NS
    epad = _pad_to(e, NW * 16 * IDXW)
    nrows_pw = epad // (NW * IDXW)
    pad = epad - e

    x_bf = x.astype(jnp.bfloat16)
    cyc = jnp.arange(pad, dtype=i32)
    ps = jnp.concatenate([pool_src.astype(i32), cyc % n])
    pd = jnp.concatenate([pool_dst.astype(i32), M_SEG + cyc % (mp - M_SEG)])
    us = jnp.concatenate([up_src.astype(i32), cyc % M_SEG])
    uk = jnp.concatenate([up_kernel.astype(i32), jnp.zeros((pad,), i32)])
    ud = jnp.concatenate([up_dst.astype(i32), n + cyc % (np_ - n)])
    src2 = ps.reshape(-1, IDXW)
    dst2 = pd.reshape(-1, IDXW)
    usrc2 = us.reshape(-1, IDXW)
    ukern2 = uk.reshape(-1, IDXW)
    udst2 = ud.reshape(-1, IDXW)

    mesh = plsc.VectorSubcoreMesh(core_axis_name="c", subcore_axis_name="s")
    sc_params = pltpu.CompilerParams(needs_layout_passes=False,
                                     use_tc_tiling_on_sc=False)
    dma_sems = [pltpu.SemaphoreType.DMA] * 6

    sums_p, counts_p = pl.kernel(
        functools.partial(_pool_body, nrows_pw, mt),
        out_type=(jax.ShapeDtypeStruct((NC, mp, ch), jnp.bfloat16),
                  jax.ShapeDtypeStruct((NW, mp), jnp.float32)),
        mesh=mesh,
        scratch_types=[
            pltpu.MemorySpace.VMEM_SHARED((mp, ch), jnp.bfloat16),
            pltpu.VMEM((2, RPC_A, IDXW), i32),
            pltpu.VMEM((2, RPC_A, IDXW), i32),
            pltpu.VMEM((2, RPC_A * IDXW, ch), jnp.bfloat16),
            pltpu.VMEM((mp,), jnp.float32),
        ] + dma_sems,
        compiler_params=sc_params,
        name="sc_pool_segment_sum",
    )(src2, dst2, x_bf)

    bm = mp // 14
    mb = mp // bm
    p4 = pl.pallas_call(
        _dense_body,
        grid=(mb,),
        in_specs=[
            pl.BlockSpec((NC, bm, ch), lambda j: (0, j, 0)),
            pl.BlockSpec((NW, bm), lambda j: (0, j)),
            pl.BlockSpec((kk, ch, ch), lambda j: (0, 0, 0)),
        ],
        out_specs=pl.BlockSpec((kk, bm, ch), lambda j: (0, j, 0)),
        out_shape=jax.ShapeDtypeStruct((kk, mp, ch), jnp.bfloat16),
    )(sums_p, counts_p, W)
    p_flat = p4.reshape(kk * mp, ch)

    up_p = pl.kernel(
        functools.partial(_up_body, nrows_pw, nt, mp),
        out_type=jax.ShapeDtypeStruct((NC, np_, ch), jnp.bfloat16),
        mesh=mesh,
        scratch_types=[
            pltpu.MemorySpace.VMEM_SHARED((np_, ch), jnp.bfloat16),
            pltpu.VMEM((2, RPC_C, IDXW), i32),
            pltpu.VMEM((2, RPC_C, IDXW), i32),
            pltpu.VMEM((2, RPC_C, IDXW), i32),
            pltpu.VMEM((2, RPC_C, IDXW), i32),
            pltpu.VMEM((2, RPC_C * IDXW, ch), jnp.bfloat16),
        ] + dma_sems,
        compiler_params=sc_params,
        name="sc_upsample_scatter",
    )(usrc2, ukern2, udst2, p_flat)

    bn = 4000
    out = pl.pallas_call(
        _final_body,
        grid=(n // bn,),
        in_specs=[
            pl.BlockSpec((bn, ch), lambda j: (j, 0)),
            pl.BlockSpec((NC, bn, ch), lambda j: (0, j, 0)),
            pl.BlockSpec((1, ch), lambda j: (0, 0)),
        ],
        out_specs=pl.BlockSpec((bn, ch), lambda j: (j, 0)),
        out_shape=jax.ShapeDtypeStruct((n, ch), jnp.float32),
    )(x, up_p, b.reshape(1, ch))
    return out
